# baseline (speedup 1.0000x reference)
import functools

import jax
import jax.numpy as jnp
from jax import lax
from jax.experimental import pallas as pl
from jax.experimental.pallas import tpu as pltpu
from jax.experimental.pallas import tpu_sc as plsc

B = 16384
D = 32
NC = 2
NS = 16
L = 16
NW = NC * NS
BPW = B // NW
G = BPW // L

_mesh = plsc.VectorSubcoreMesh(core_axis_name="c", subcore_axis_name="s")


@functools.partial(
    pl.kernel,
    mesh=_mesh,
    out_type=jax.ShapeDtypeStruct((B,), jnp.float32),
    scratch_types=[
        pltpu.VMEM((BPW,), jnp.int32),
        pltpu.VMEM((BPW,), jnp.int32),
        pltpu.VMEM((BPW, D), jnp.float32),
        pltpu.VMEM((BPW, D), jnp.float32),
        pltpu.VMEM((BPW,), jnp.float32),
        pltpu.SemaphoreType.DMA,
    ],
    compiler_params=pltpu.CompilerParams(
        needs_layout_passes=False, use_tc_tiling_on_sc=False),
)
def _mf_dot(u_hbm, v_hbm, ue_hbm, ve_hbm, out_hbm,
            idxu, idxv, rows_u, rows_v, outv, sem):
    wid = lax.axis_index("s") * NC + lax.axis_index("c")
    base = wid * BPW

    pltpu.sync_copy(u_hbm.at[pl.ds(base, BPW)], idxu)
    pltpu.sync_copy(v_hbm.at[pl.ds(base, BPW)], idxv)

    cu = pltpu.async_copy(ue_hbm.at[idxu], rows_u, sem)
    cv = pltpu.async_copy(ve_hbm.at[idxv], rows_v, sem)
    cu.wait()
    cv.wait()

    lane = lax.iota(jnp.int32, L)

    def group(g, carry):
        row = g * L + lane
        acc = jnp.zeros((L,), jnp.float32)
        for d in range(D):
            col = jnp.full((L,), d, jnp.int32)
            xu = plsc.load_gather(rows_u, [row, col])
            xv = plsc.load_gather(rows_v, [row, col])
            acc = acc + xu * xv
        outv[pl.ds(g * L, L)] = acc
        return carry

    lax.fori_loop(0, G, group, 0)

    pltpu.sync_copy(outv, out_hbm.at[pl.ds(base, BPW)])


def kernel(u, v, user_emb, item_emb):
    return _mf_dot(u.astype(jnp.int32), v.astype(jnp.int32),
                   user_emb, item_emb)
